# baseline (speedup 1.0000x reference)
import functools

import jax
import jax.numpy as jnp
from jax import lax
from jax.experimental import pallas as pl
from jax.experimental.pallas import tpu as pltpu
from jax.experimental.pallas import tpu_sc as plsc

_B = 2
_C = 3
_N = 4096
_LAT = 128
_KQ = 132
_KT = 83
_KP = 96
_NV = _KP // 16
_TAIL = _KT - 5 * 16

_NW = 32
_RPW = (_B * _N) // _NW
_BR = 8
_NBLK = _RPW // _BR

_TQ = 256
_TM = 256


def _proj_body(x_ref, tmb_ref, wb_ref, bb_ref, wk_ref, bk_ref, wq_ref, bq_ref,
               mfeat_ref, ks_ref, qs_ref):
    x = x_ref[0]
    feat = jnp.dot(x, wb_ref[...], preferred_element_type=jnp.float32) + bb_ref[...]
    mfeat_ref[0] = feat * (tmb_ref[0] > 0).astype(jnp.float32)
    ks_ref[0] = jnp.dot(feat, wk_ref[...], preferred_element_type=jnp.float32) + bk_ref[...]
    qs_ref[0] = jnp.dot(feat, wq_ref[...], preferred_element_type=jnp.float32) + bq_ref[...]


def _stage_a(xp, tmb, wbp, bb2, wk, bk2, wq, bq2):
    return pl.pallas_call(
        _proj_body,
        grid=(_B,),
        in_specs=[
            pl.BlockSpec((1, _N, _LAT), lambda b: (b, 0, 0)),
            pl.BlockSpec((1, _N, _LAT), lambda b: (b, 0, 0)),
            pl.BlockSpec((_LAT, _LAT), lambda b: (0, 0)),
            pl.BlockSpec((1, _LAT), lambda b: (0, 0)),
            pl.BlockSpec((_LAT, _KQ), lambda b: (0, 0)),
            pl.BlockSpec((1, _KQ), lambda b: (0, 0)),
            pl.BlockSpec((_LAT, _KQ), lambda b: (0, 0)),
            pl.BlockSpec((1, _KQ), lambda b: (0, 0)),
        ],
        out_specs=[
            pl.BlockSpec((1, _N, _LAT), lambda b: (b, 0, 0)),
            pl.BlockSpec((1, _N, _KQ), lambda b: (b, 0, 0)),
            pl.BlockSpec((1, _N, _KQ), lambda b: (b, 0, 0)),
        ],
        out_shape=[
            jax.ShapeDtypeStruct((_B, _N, _LAT), jnp.float32),
            jax.ShapeDtypeStruct((_B, _N, _KQ), jnp.float32),
            jax.ShapeDtypeStruct((_B, _N, _KQ), jnp.float32),
        ],
    )(xp, tmb, wbp, bb2, wk, bk2, wq, bq2)


def _logits_body(qs_ref, ks_ref, l_ref):
    l_ref[0] = lax.dot_general(
        qs_ref[0], ks_ref[0], (((1,), (1,)), ((), ())),
        preferred_element_type=jnp.float32)


def _stage_b(qs, ks):
    return pl.pallas_call(
        _logits_body,
        grid=(_B, _N // _TQ),
        in_specs=[
            pl.BlockSpec((1, _TQ, _KQ), lambda b, t: (b, t, 0)),
            pl.BlockSpec((1, _N, _KQ), lambda b, t: (b, 0, 0)),
        ],
        out_specs=pl.BlockSpec((1, _TQ, _N), lambda b, t: (b, t, 0)),
        out_shape=jax.ShapeDtypeStruct((_B, _N, _N), jnp.float32),
    )(qs, ks)


_SC_MESH = plsc.VectorSubcoreMesh(core_axis_name="c", subcore_axis_name="s")


@functools.partial(
    pl.kernel,
    out_type=jax.ShapeDtypeStruct((_B * _N, _N), jnp.float32),
    mesh=_SC_MESH,
    compiler_params=pltpu.CompilerParams(needs_layout_passes=False),
    scratch_types=[
        pltpu.VMEM((_BR, _KP), jnp.int32),
        pltpu.VMEM((_BR, _KP), jnp.int32),
        pltpu.VMEM((_BR * _N,), jnp.float32),
        pltpu.VMEM((_BR * _N,), jnp.float32),
        pltpu.VMEM((_BR * _N,), jnp.float32),
        pltpu.SemaphoreType.DMA,
        pltpu.SemaphoreType.DMA,
        pltpu.SemaphoreType.DMA,
    ],
)
def _stage_c(l_hbm, vcol_hbm, w_hbm, idxc0, idxc1, lbuf0, lbuf1,
             wrow, insem0, insem1, outsem):
    wid = lax.axis_index("s") * 2 + lax.axis_index("c")
    base = wid * _RPW
    idxc = (idxc0, idxc1)
    lbuf = (lbuf0, lbuf1)
    insem = (insem0, insem1)

    zero16 = jnp.zeros((16,), jnp.float32)
    lane = lax.broadcasted_iota(jnp.int32, (16,), 0)
    tail_valid = lane < _TAIL

    def _z(i, _):
        wrow[pl.ds(i * 16, 16)] = zero16
        return 0
    lax.fori_loop(0, _BR * _N // 16, _z, 0)

    def _in_copies(blk, u):
        rowstart = base + blk * _BR
        return [pltpu.make_async_copy(l_hbm.at[rowstart + j],
                                      lbuf[u].at[pl.ds(j * _N, _N)], insem[u])
                for j in range(_BR)]

    def _out_copies(blk):
        rowstart = base + blk * _BR
        return [pltpu.make_async_copy(wrow.at[pl.ds(j * _N, _N)],
                                      w_hbm.at[rowstart + j], outsem)
                for j in range(_BR)]

    def _start_in(blk, u):
        for d in _in_copies(blk, u):
            d.start()
        pltpu.sync_copy(vcol_hbm.at[pl.ds(base + blk * _BR, _BR)], idxc[u])

    def _do_block(blk, u):
        for d in _in_copies(blk, u):
            d.wait()

        @pl.when(blk > 0)
        def _():
            for d in _out_copies(blk - 1):
                d.wait()

            def _rz(j, _):
                for g in range(_NV):
                    cg = idxc[1 - u][j, pl.ds(g * 16, 16)] + j * _N
                    msk = None if g < _NV - 1 else tail_valid
                    plsc.store_scatter(wrow, [cg], zero16, mask=msk)
                return 0
            lax.fori_loop(0, _BR, _rz, 0)

        @pl.when(blk + 1 < _NBLK)
        def _():
            _start_in(blk + 1, 1 - u)

        def _row(j, _):
            cols = [idxc[u][j, pl.ds(g * 16, 16)] + j * _N for g in range(_NV)]
            logit = [plsc.load_gather(lbuf[u], [cg]) for cg in cols]
            logit[_NV - 1] = jnp.where(tail_valid, logit[_NV - 1], -1e30)
            m = logit[0]
            for g in range(1, _NV):
                m = jnp.maximum(m, logit[g])
            mx = jnp.max(m)
            e = [jnp.exp(lg - mx) for lg in logit]
            acc = e[0]
            for g in range(1, _NV):
                acc = acc + e[g]
            z = jnp.sum(acc)
            denom = jnp.full((16,), 1e-12, jnp.float32) + z
            s = jnp.full((16,), 1.0, jnp.float32) / denom
            for g in range(_NV):
                w = e[g] * s
                if g < _NV - 1:
                    plsc.addupdate_scatter(wrow, [cols[g]], w)
                else:
                    plsc.addupdate_scatter(wrow, [cols[g]], w, mask=tail_valid)
            return 0

        lax.fori_loop(0, _BR, _row, 0)
        for d in _out_copies(blk):
            d.start()

    _start_in(0, 0)

    def _pair(i, _):
        _do_block(i * 2, 0)
        _do_block(i * 2 + 1, 1)
        return 0

    lax.fori_loop(0, _NBLK // 2, _pair, 0)
    for d in _out_copies(_NBLK - 1):
        d.wait()


def _out_body(w_ref, mf_ref, out_ref):
    out_ref[0] = lax.dot_general(
        w_ref[0], mf_ref[0], (((0,), (0,)), ((), ())),
        preferred_element_type=jnp.float32)


def _stage_d(wmat, mfeat):
    return pl.pallas_call(
        _out_body,
        grid=(_B, _N // _TM),
        in_specs=[
            pl.BlockSpec((1, _N, _TM), lambda b, t: (b, 0, t)),
            pl.BlockSpec((1, _N, _LAT), lambda b, t: (b, 0, 0)),
        ],
        out_specs=pl.BlockSpec((1, _TM, _LAT), lambda b, t: (b, t, 0)),
        out_shape=jax.ShapeDtypeStruct((_B, _N, _LAT), jnp.float32),
    )(wmat, mfeat)


def kernel(ims, target_masks, Wb, bb, Wk, bk, Wq, bq, local_inds, long_inds):
    b, c, h, w = ims.shape
    n = h * w
    x = ims.reshape(b, c, n).transpose(0, 2, 1)
    xp = jnp.pad(x, ((0, 0), (0, 0), (0, _LAT - c)))
    wbp = jnp.pad(Wb, ((0, _LAT - c), (0, 0)))
    tmb = jnp.broadcast_to(target_masks.reshape(b, n, 1), (b, n, _LAT))
    bb2 = bb.reshape(1, _LAT)
    bk2 = bk.reshape(1, _KQ)
    bq2 = bq.reshape(1, _KQ)

    v = jnp.concatenate(
        [jnp.broadcast_to(local_inds[None].astype(jnp.int32),
                          (b, n, local_inds.shape[1])),
         long_inds.astype(jnp.int32)], axis=-1)
    vp = jnp.concatenate(
        [v, jnp.broadcast_to(v[..., :1], (b, n, _KP - _KT))], axis=-1)
    vcol = vp.reshape(b * n, _KP)

    mfeat, ks, qs = _stage_a(xp, tmb, wbp, bb2, Wk, bk2, Wq, bq2)
    logits = _stage_b(qs, ks)
    wmat = _stage_c(logits.reshape(b * n, n), vcol)
    out = _stage_d(wmat.reshape(b, n, n), mfeat)
    return out


# speedup vs baseline: 1.0729x; 1.0729x over previous
import functools

import jax
import jax.numpy as jnp
from jax import lax
from jax.experimental import pallas as pl
from jax.experimental.pallas import tpu as pltpu
from jax.experimental.pallas import tpu_sc as plsc

_B = 2
_C = 3
_N = 4096
_LAT = 128
_KQ = 132
_KT = 83
_KP = 96
_NV = _KP // 16
_TAIL = _KT - 5 * 16

_NW = 32
_RPW = (_B * _N) // _NW
_BR = 8
_NBLK = _RPW // _BR

_TQ = 256
_TM = 256


def _proj_body(x_ref, tmb_ref, wb_ref, bb_ref, wk_ref, bk_ref, wq_ref, bq_ref,
               mfeat_ref, ks_ref, qs_ref):
    x = x_ref[0]
    feat = jnp.dot(x, wb_ref[...], preferred_element_type=jnp.float32) + bb_ref[...]
    mfeat_ref[0] = feat * (tmb_ref[0] > 0).astype(jnp.float32)
    ks_ref[0] = jnp.dot(feat, wk_ref[...], preferred_element_type=jnp.float32) + bk_ref[...]
    qs_ref[0] = jnp.dot(feat, wq_ref[...], preferred_element_type=jnp.float32) + bq_ref[...]


def _stage_a(xp, tmb, wbp, bb2, wk, bk2, wq, bq2):
    return pl.pallas_call(
        _proj_body,
        grid=(_B,),
        in_specs=[
            pl.BlockSpec((1, _N, _LAT), lambda b: (b, 0, 0)),
            pl.BlockSpec((1, _N, _LAT), lambda b: (b, 0, 0)),
            pl.BlockSpec((_LAT, _LAT), lambda b: (0, 0)),
            pl.BlockSpec((1, _LAT), lambda b: (0, 0)),
            pl.BlockSpec((_LAT, _KQ), lambda b: (0, 0)),
            pl.BlockSpec((1, _KQ), lambda b: (0, 0)),
            pl.BlockSpec((_LAT, _KQ), lambda b: (0, 0)),
            pl.BlockSpec((1, _KQ), lambda b: (0, 0)),
        ],
        out_specs=[
            pl.BlockSpec((1, _N, _LAT), lambda b: (b, 0, 0)),
            pl.BlockSpec((1, _N, _KQ), lambda b: (b, 0, 0)),
            pl.BlockSpec((1, _N, _KQ), lambda b: (b, 0, 0)),
        ],
        out_shape=[
            jax.ShapeDtypeStruct((_B, _N, _LAT), jnp.float32),
            jax.ShapeDtypeStruct((_B, _N, _KQ), jnp.float32),
            jax.ShapeDtypeStruct((_B, _N, _KQ), jnp.float32),
        ],
    )(xp, tmb, wbp, bb2, wk, bk2, wq, bq2)


def _logits_body(qs_ref, ks_ref, l_ref):
    l = lax.dot_general(
        qs_ref[0], ks_ref[0], (((1,), (1,)), ((), ())),
        preferred_element_type=jnp.float32)
    lo = lax.bitcast_convert_type(
        l[:, : _N // 2].astype(jnp.bfloat16), jnp.uint16).astype(jnp.int32)
    hi = lax.bitcast_convert_type(
        l[:, _N // 2:].astype(jnp.bfloat16), jnp.uint16).astype(jnp.int32)
    l_ref[0] = (hi << 16) | lo


def _stage_b(qs, ks):
    return pl.pallas_call(
        _logits_body,
        grid=(_B, _N // _TQ),
        in_specs=[
            pl.BlockSpec((1, _TQ, _KQ), lambda b, t: (b, t, 0)),
            pl.BlockSpec((1, _N, _KQ), lambda b, t: (b, 0, 0)),
        ],
        out_specs=pl.BlockSpec((1, _TQ, _N // 2), lambda b, t: (b, t, 0)),
        out_shape=jax.ShapeDtypeStruct((_B, _N, _N // 2), jnp.int32),
    )(qs, ks)


_SC_MESH = plsc.VectorSubcoreMesh(core_axis_name="c", subcore_axis_name="s")


@functools.partial(
    pl.kernel,
    out_type=jax.ShapeDtypeStruct((_B * _N, _N), jnp.float32),
    mesh=_SC_MESH,
    compiler_params=pltpu.CompilerParams(needs_layout_passes=False),
    scratch_types=[
        pltpu.VMEM((_BR, _KP), jnp.int32),
        pltpu.VMEM((_BR, _KP), jnp.int32),
        pltpu.VMEM((_BR * _N // 2,), jnp.int32),
        pltpu.VMEM((_BR * _N // 2,), jnp.int32),
        pltpu.VMEM((_BR * _N,), jnp.float32),
        pltpu.SemaphoreType.DMA,
        pltpu.SemaphoreType.DMA,
        pltpu.SemaphoreType.DMA,
    ],
)
def _stage_c(l_hbm, vcol_hbm, w_hbm, idxc0, idxc1, lbuf0, lbuf1,
             wrow, insem0, insem1, outsem):
    wid = lax.axis_index("s") * 2 + lax.axis_index("c")
    base = wid * _RPW
    idxc = (idxc0, idxc1)
    lbuf = (lbuf0, lbuf1)
    insem = (insem0, insem1)

    zero16 = jnp.zeros((16,), jnp.float32)
    lane = lax.broadcasted_iota(jnp.int32, (16,), 0)
    tail_valid = lane < _TAIL

    def _z(i, _):
        wrow[pl.ds(i * 16, 16)] = zero16
        return 0
    lax.fori_loop(0, _BR * _N // 16, _z, 0)

    def _in_copies(blk, u):
        rowstart = base + blk * _BR
        return [pltpu.make_async_copy(l_hbm.at[rowstart + j],
                                      lbuf[u].at[pl.ds(j * _N // 2, _N // 2)],
                                      insem[u])
                for j in range(_BR)]

    def _out_copies(blk):
        rowstart = base + blk * _BR
        return [pltpu.make_async_copy(wrow.at[pl.ds(j * _N, _N)],
                                      w_hbm.at[rowstart + j], outsem)
                for j in range(_BR)]

    def _start_in(blk, u):
        for d in _in_copies(blk, u):
            d.start()
        pltpu.sync_copy(vcol_hbm.at[pl.ds(base + blk * _BR, _BR)], idxc[u])

    def _do_block(blk, u):
        for d in _in_copies(blk, u):
            d.wait()

        @pl.when(blk > 0)
        def _():
            for d in _out_copies(blk - 1):
                d.wait()

            def _rz(j, _):
                for g in range(_NV):
                    cg = idxc[1 - u][j, pl.ds(g * 16, 16)] + j * _N
                    msk = None if g < _NV - 1 else tail_valid
                    plsc.store_scatter(wrow, [cg], zero16, mask=msk)
                return 0
            lax.fori_loop(0, _BR, _rz, 0)

        @pl.when(blk + 1 < _NBLK)
        def _():
            _start_in(blk + 1, 1 - u)

        def _row(j, _):
            vcols = [idxc[u][j, pl.ds(g * 16, 16)] for g in range(_NV)]
            cols = [vc + j * _N for vc in vcols]
            logit = []
            for g in range(_NV):
                widx = (vcols[g] & (_N // 2 - 1)) + j * (_N // 2)
                u32 = plsc.load_gather(lbuf[u], [widx])
                hi = vcols[g] >= (_N // 2)
                bits = jnp.where(hi, u32 & jnp.int32(-65536), u32 << 16)
                logit.append(lax.bitcast_convert_type(bits, jnp.float32))
            logit[_NV - 1] = jnp.where(tail_valid, logit[_NV - 1], -1e30)
            m = logit[0]
            for g in range(1, _NV):
                m = jnp.maximum(m, logit[g])
            mx = jnp.max(m)
            e = [jnp.exp(lg - mx) for lg in logit]
            acc = e[0]
            for g in range(1, _NV):
                acc = acc + e[g]
            z = jnp.sum(acc)
            denom = jnp.full((16,), 1e-12, jnp.float32) + z
            s = jnp.full((16,), 1.0, jnp.float32) / denom
            for g in range(_NV):
                w = e[g] * s
                if g < _NV - 1:
                    plsc.addupdate_scatter(wrow, [cols[g]], w)
                else:
                    plsc.addupdate_scatter(wrow, [cols[g]], w, mask=tail_valid)
            return 0

        lax.fori_loop(0, _BR, _row, 0)
        for d in _out_copies(blk):
            d.start()

    _start_in(0, 0)

    def _pair(i, _):
        _do_block(i * 2, 0)
        _do_block(i * 2 + 1, 1)
        return 0

    lax.fori_loop(0, _NBLK // 2, _pair, 0)
    for d in _out_copies(_NBLK - 1):
        d.wait()


def _out_body(w_ref, mf_ref, out_ref):
    out_ref[0] = lax.dot_general(
        w_ref[0], mf_ref[0], (((0,), (0,)), ((), ())),
        preferred_element_type=jnp.float32)


def _stage_d(wmat, mfeat):
    return pl.pallas_call(
        _out_body,
        grid=(_B, _N // _TM),
        in_specs=[
            pl.BlockSpec((1, _N, _TM), lambda b, t: (b, 0, t)),
            pl.BlockSpec((1, _N, _LAT), lambda b, t: (b, 0, 0)),
        ],
        out_specs=pl.BlockSpec((1, _TM, _LAT), lambda b, t: (b, t, 0)),
        out_shape=jax.ShapeDtypeStruct((_B, _N, _LAT), jnp.float32),
    )(wmat, mfeat)


def kernel(ims, target_masks, Wb, bb, Wk, bk, Wq, bq, local_inds, long_inds):
    b, c, h, w = ims.shape
    n = h * w
    x = ims.reshape(b, c, n).transpose(0, 2, 1)
    xp = jnp.pad(x, ((0, 0), (0, 0), (0, _LAT - c)))
    wbp = jnp.pad(Wb, ((0, _LAT - c), (0, 0)))
    tmb = jnp.broadcast_to(target_masks.reshape(b, n, 1), (b, n, _LAT))
    bb2 = bb.reshape(1, _LAT)
    bk2 = bk.reshape(1, _KQ)
    bq2 = bq.reshape(1, _KQ)

    v = jnp.concatenate(
        [jnp.broadcast_to(local_inds[None].astype(jnp.int32),
                          (b, n, local_inds.shape[1])),
         long_inds.astype(jnp.int32)], axis=-1)
    vp = jnp.concatenate(
        [v, jnp.broadcast_to(v[..., :1], (b, n, _KP - _KT))], axis=-1)
    vcol = vp.reshape(b * n, _KP)

    mfeat, ks, qs = _stage_a(xp, tmb, wbp, bb2, Wk, bk2, Wq, bq2)
    logits = _stage_b(qs, ks)
    wmat = _stage_c(logits.reshape(b * n, n // 2), vcol)
    out = _stage_d(wmat.reshape(b, n, n), mfeat)
    return out


# speedup vs baseline: 1.1038x; 1.0288x over previous
import functools

import jax
import jax.numpy as jnp
from jax import lax
from jax.experimental import pallas as pl
from jax.experimental.pallas import tpu as pltpu
from jax.experimental.pallas import tpu_sc as plsc

_B = 2
_C = 3
_N = 4096
_LAT = 128
_KQ = 132
_KT = 83
_KP = 96
_NV = _KP // 16
_TAIL = _KT - 5 * 16

_NW = 32
_RPW = _N // _NW
_BR = 8
_NBLK = _RPW // _BR

_TQ = 256
_TM = 256


def _proj_body(x_ref, tmb_ref, wb_ref, bb_ref, wk_ref, bk_ref, wq_ref, bq_ref,
               mfeat_ref, ks_ref, qs_ref):
    x = x_ref[0]
    feat = jnp.dot(x, wb_ref[...], preferred_element_type=jnp.float32) + bb_ref[...]
    mfeat_ref[0] = feat * (tmb_ref[0] > 0).astype(jnp.float32)
    ks_ref[0] = jnp.dot(feat, wk_ref[...], preferred_element_type=jnp.float32) + bk_ref[...]
    qs_ref[0] = jnp.dot(feat, wq_ref[...], preferred_element_type=jnp.float32) + bq_ref[...]


def _stage_a(xp, tmb, wbp, bb2, wk, bk2, wq, bq2):
    return pl.pallas_call(
        _proj_body,
        grid=(_B,),
        in_specs=[
            pl.BlockSpec((1, _N, _LAT), lambda b: (b, 0, 0)),
            pl.BlockSpec((1, _N, _LAT), lambda b: (b, 0, 0)),
            pl.BlockSpec((_LAT, _LAT), lambda b: (0, 0)),
            pl.BlockSpec((1, _LAT), lambda b: (0, 0)),
            pl.BlockSpec((_LAT, _KQ), lambda b: (0, 0)),
            pl.BlockSpec((1, _KQ), lambda b: (0, 0)),
            pl.BlockSpec((_LAT, _KQ), lambda b: (0, 0)),
            pl.BlockSpec((1, _KQ), lambda b: (0, 0)),
        ],
        out_specs=[
            pl.BlockSpec((1, _N, _LAT), lambda b: (b, 0, 0)),
            pl.BlockSpec((1, _N, _KQ), lambda b: (b, 0, 0)),
            pl.BlockSpec((1, _N, _KQ), lambda b: (b, 0, 0)),
        ],
        out_shape=[
            jax.ShapeDtypeStruct((_B, _N, _LAT), jnp.float32),
            jax.ShapeDtypeStruct((_B, _N, _KQ), jnp.float32),
            jax.ShapeDtypeStruct((_B, _N, _KQ), jnp.float32),
        ],
    )(xp, tmb, wbp, bb2, wk, bk2, wq, bq2)


def _logits_body(qs_ref, ks_ref, l_ref):
    l = lax.dot_general(
        qs_ref[...], ks_ref[...], (((1,), (1,)), ((), ())),
        preferred_element_type=jnp.float32)
    lo = lax.bitcast_convert_type(
        l[:, : _N // 2].astype(jnp.bfloat16), jnp.uint16).astype(jnp.int32)
    hi = lax.bitcast_convert_type(
        l[:, _N // 2:].astype(jnp.bfloat16), jnp.uint16).astype(jnp.int32)
    l_ref[...] = (hi << 16) | lo


def _stage_b(qs, ks):
    return pl.pallas_call(
        _logits_body,
        grid=(_N // _TQ,),
        in_specs=[
            pl.BlockSpec((_TQ, _KQ), lambda t: (t, 0)),
            pl.BlockSpec((_N, _KQ), lambda t: (0, 0)),
        ],
        out_specs=pl.BlockSpec((_TQ, _N // 2), lambda t: (t, 0)),
        out_shape=jax.ShapeDtypeStruct((_N, _N // 2), jnp.int32),
    )(qs, ks)


_SC_MESH = plsc.VectorSubcoreMesh(core_axis_name="c", subcore_axis_name="s")


@functools.partial(
    pl.kernel,
    out_type=jax.ShapeDtypeStruct((_N, _N), jnp.float32),
    mesh=_SC_MESH,
    compiler_params=pltpu.CompilerParams(needs_layout_passes=False),
    scratch_types=[
        pltpu.VMEM((_BR, _KP), jnp.int32),
        pltpu.VMEM((_BR, _KP), jnp.int32),
        pltpu.VMEM((_BR * _N // 2,), jnp.int32),
        pltpu.VMEM((_BR * _N // 2,), jnp.int32),
        pltpu.VMEM((_BR * _N,), jnp.float32),
        pltpu.SemaphoreType.DMA,
        pltpu.SemaphoreType.DMA,
        pltpu.SemaphoreType.DMA,
    ],
)
def _stage_c(l_hbm, vcol_hbm, w_hbm, idxc0, idxc1, lbuf0, lbuf1,
             wrow, insem0, insem1, outsem):
    wid = lax.axis_index("s") * 2 + lax.axis_index("c")
    base = wid * _RPW
    idxc = (idxc0, idxc1)
    lbuf = (lbuf0, lbuf1)
    insem = (insem0, insem1)

    zero16 = jnp.zeros((16,), jnp.float32)
    lane = lax.broadcasted_iota(jnp.int32, (16,), 0)
    tail_valid = lane < _TAIL

    def _z(i, _):
        wrow[pl.ds(i * 16, 16)] = zero16
        return 0
    lax.fori_loop(0, _BR * _N // 16, _z, 0)

    def _in_copies(blk, u):
        rowstart = base + blk * _BR
        return [pltpu.make_async_copy(l_hbm.at[rowstart + j],
                                      lbuf[u].at[pl.ds(j * _N // 2, _N // 2)],
                                      insem[u])
                for j in range(_BR)]

    def _out_copies(blk):
        rowstart = base + blk * _BR
        return [pltpu.make_async_copy(wrow.at[pl.ds(j * _N, _N)],
                                      w_hbm.at[rowstart + j], outsem)
                for j in range(_BR)]

    def _start_in(blk, u):
        for d in _in_copies(blk, u):
            d.start()
        pltpu.sync_copy(vcol_hbm.at[pl.ds(base + blk * _BR, _BR)], idxc[u])

    def _do_block(blk, u):
        for d in _in_copies(blk, u):
            d.wait()

        @pl.when(blk > 0)
        def _():
            for d in _out_copies(blk - 1):
                d.wait()

            def _rz(j, _):
                for g in range(_NV):
                    cg = idxc[1 - u][j, pl.ds(g * 16, 16)] + j * _N
                    msk = None if g < _NV - 1 else tail_valid
                    plsc.store_scatter(wrow, [cg], zero16, mask=msk)
                return 0
            lax.fori_loop(0, _BR, _rz, 0)

        @pl.when(blk + 1 < _NBLK)
        def _():
            _start_in(blk + 1, 1 - u)

        def _row(j, _):
            vcols = [idxc[u][j, pl.ds(g * 16, 16)] for g in range(_NV)]
            cols = [vc + j * _N for vc in vcols]
            logit = []
            for g in range(_NV):
                widx = (vcols[g] & (_N // 2 - 1)) + j * (_N // 2)
                u32 = plsc.load_gather(lbuf[u], [widx])
                hi = vcols[g] >= (_N // 2)
                bits = jnp.where(hi, u32 & jnp.int32(-65536), u32 << 16)
                logit.append(lax.bitcast_convert_type(bits, jnp.float32))
            logit[_NV - 1] = jnp.where(tail_valid, logit[_NV - 1], -1e30)
            m = logit[0]
            for g in range(1, _NV):
                m = jnp.maximum(m, logit[g])
            mx = jnp.max(m)
            e = [jnp.exp(lg - mx) for lg in logit]
            acc = e[0]
            for g in range(1, _NV):
                acc = acc + e[g]
            z = jnp.sum(acc)
            denom = jnp.full((16,), 1e-12, jnp.float32) + z
            s = jnp.full((16,), 1.0, jnp.float32) / denom
            for g in range(_NV):
                w = e[g] * s
                if g < _NV - 1:
                    plsc.addupdate_scatter(wrow, [cols[g]], w)
                else:
                    plsc.addupdate_scatter(wrow, [cols[g]], w, mask=tail_valid)
            return 0

        lax.fori_loop(0, _BR, _row, 0)
        for d in _out_copies(blk):
            d.start()

    _start_in(0, 0)

    def _pair(i, _):
        _do_block(i * 2, 0)
        _do_block(i * 2 + 1, 1)
        return 0

    lax.fori_loop(0, _NBLK // 2, _pair, 0)
    for d in _out_copies(_NBLK - 1):
        d.wait()


def _out_body(w_ref, mf_ref, out_ref):
    out_ref[...] = lax.dot_general(
        w_ref[...], mf_ref[...], (((0,), (0,)), ((), ())),
        preferred_element_type=jnp.float32)


def _stage_d(wmat, mfeat):
    return pl.pallas_call(
        _out_body,
        grid=(_N // _TM,),
        in_specs=[
            pl.BlockSpec((_N, _TM), lambda t: (0, t)),
            pl.BlockSpec((_N, _LAT), lambda t: (0, 0)),
        ],
        out_specs=pl.BlockSpec((_TM, _LAT), lambda t: (t, 0)),
        out_shape=jax.ShapeDtypeStruct((_N, _LAT), jnp.float32),
    )(wmat, mfeat)


def kernel(ims, target_masks, Wb, bb, Wk, bk, Wq, bq, local_inds, long_inds):
    b, c, h, w = ims.shape
    n = h * w
    x = ims.reshape(b, c, n).transpose(0, 2, 1)
    xp = jnp.pad(x, ((0, 0), (0, 0), (0, _LAT - c)))
    wbp = jnp.pad(Wb, ((0, _LAT - c), (0, 0)))
    tmb = jnp.broadcast_to(target_masks.reshape(b, n, 1), (b, n, _LAT))
    bb2 = bb.reshape(1, _LAT)
    bk2 = bk.reshape(1, _KQ)
    bq2 = bq.reshape(1, _KQ)

    v = jnp.concatenate(
        [jnp.broadcast_to(local_inds[None].astype(jnp.int32),
                          (b, n, local_inds.shape[1])),
         long_inds.astype(jnp.int32)], axis=-1)
    vp = jnp.concatenate(
        [v, jnp.broadcast_to(v[..., :1], (b, n, _KP - _KT))], axis=-1)
    vcol = vp.reshape(b * n, _KP)

    mfeat, ks, qs = _stage_a(xp, tmb, wbp, bb2, Wk, bk2, Wq, bq2)
    outs = []
    for bi in range(b):
        logits = _stage_b(qs[bi], ks[bi])
        wmat = _stage_c(logits, vcol[bi * n:(bi + 1) * n])
        outs.append(_stage_d(wmat, mfeat[bi]))
    return jnp.stack(outs)
